# Initial kernel scaffold; baseline (speedup 1.0000x reference)
import jax
import jax.numpy as jnp
from jax.experimental import pallas as pl


def kernel(edge_index, user_emb, item_emb):
    raise NotImplementedError("write your pallas kernel here")


# speedup vs baseline: 10.8722x; 10.8722x over previous
import functools

import jax
import jax.numpy as jnp
from jax import lax
from jax.experimental import pallas as pl
from jax.experimental.pallas import tpu as pltpu
from jax.experimental.pallas import tpu_sc as plsc

NU = 50000
NT = 2 * NU
D = 64
H = 32
E = 800000
NC, NS, LN = 2, 16, 16
CH = 128
EP = 802816
G = EP // NS // CH
J = 8
NG = G // J
R = 51200
RT = R // NS
TRASH = R - 1
ZR = 400

_mesh = plsc.VectorSubcoreMesh(
    core_axis_name="c", subcore_axis_name="s", num_cores=NC, num_subcores=NS
)
_params = pltpu.CompilerParams(use_tc_tiling_on_sc=False)


def _zero_f32(ref, nrows, ncols):
    z = jnp.zeros((LN,), jnp.float32)

    def body(r, _):
        for c0 in range(0, ncols, LN):
            ref[r, pl.ds(c0, LN)] = z
        return 0

    lax.fori_loop(0, nrows, body, 0)


def _zero_f32_1d(ref, n):
    z = jnp.zeros((LN,), jnp.float32)

    def body(r, _):
        ref[pl.ds(r * LN, LN)] = z
        return 0

    lax.fori_loop(0, n // LN, body, 0)


def _hist_body(dst3, cnt, didx, ones, zbuf, acc):
    c = lax.axis_index("c")
    s = lax.axis_index("s")
    one = jnp.ones((LN,), jnp.float32)
    for c0 in range(0, CH, LN):
        ones[pl.ds(c0, LN)] = one
    _zero_f32_1d(zbuf, RT)
    pltpu.sync_copy(zbuf, acc.at[pl.ds(s * RT, RT)])
    plsc.subcore_barrier()

    def group(g, _):
        base = s * G + g * J
        pltpu.sync_copy(dst3.at[c, pl.ds(base, J)], didx)
        for j in range(J):
            pltpu.sync_copy(ones, acc.at[didx.at[j]], add=True)
        return 0

    lax.fori_loop(0, NG, group, 0)
    plsc.subcore_barrier()
    pltpu.sync_copy(acc.at[pl.ds(s * RT, RT)], cnt.at[c, pl.ds(s * RT, RT)])


_hist = pl.kernel(
    _hist_body,
    out_type=jax.ShapeDtypeStruct((NC, R), jnp.float32),
    mesh=_mesh,
    scratch_types=[
        pltpu.VMEM((J, CH), jnp.int32),
        pltpu.VMEM((CH,), jnp.float32),
        pltpu.VMEM((RT,), jnp.float32),
        pltpu.VMEM_SHARED((R,), jnp.float32),
    ],
    compiler_params=_params,
)


def _layer_body(src3, dst3, y0, y1, o0, o1, sidx, didx, rows, zbuf, acc, sem):
    c = lax.axis_index("c")
    s = lax.axis_index("s")
    _zero_f32(zbuf, ZR, H)
    for h, (y, o) in enumerate(((y0, o0), (y1, o1))):
        for k in range(RT // ZR):
            pltpu.sync_copy(zbuf, acc.at[pl.ds(s * RT + k * ZR, ZR)])
        plsc.subcore_barrier()

        def group(g, _):
            base = s * G + g * J
            pltpu.sync_copy(src3.at[c, pl.ds(base, J)], sidx)
            pltpu.sync_copy(dst3.at[c, pl.ds(base, J)], didx)
            for j in range(J):
                pltpu.async_copy(y.at[sidx.at[j]], rows, sem).wait()
                pltpu.sync_copy(rows, acc.at[didx.at[j]], add=True)
            return 0

        lax.fori_loop(0, NG, group, 0)
        plsc.subcore_barrier()
        pltpu.sync_copy(
            acc.at[pl.ds(s * RT, RT)], o.at[c, pl.ds(s * RT, RT)]
        )
        if h == 0:
            plsc.subcore_barrier()


_layer = pl.kernel(
    _layer_body,
    out_type=[
        jax.ShapeDtypeStruct((NC, R, H), jnp.float32),
        jax.ShapeDtypeStruct((NC, R, H), jnp.float32),
    ],
    mesh=_mesh,
    scratch_types=[
        pltpu.VMEM((J, CH), jnp.int32),
        pltpu.VMEM((J, CH), jnp.int32),
        pltpu.VMEM((CH, H), jnp.float32),
        pltpu.VMEM((ZR, H), jnp.float32),
        pltpu.VMEM_SHARED((R, H), jnp.float32),
        pltpu.SemaphoreType.DMA,
    ],
    compiler_params=_params,
)


@jax.jit
def kernel(edge_index, user_emb, item_emb):
    edge_index = edge_index.astype(jnp.int32)
    u, it = edge_index[0], edge_index[1]
    pad = EP - E
    dst = jnp.stack([u, it])
    src = jnp.stack([it + NU, u])
    dst3 = jnp.pad(dst, ((0, 0), (0, pad)), constant_values=TRASH)
    src3 = jnp.pad(src, ((0, 0), (0, pad)), constant_values=0)
    dst3 = dst3.reshape(NC, EP // CH, CH)
    src3 = src3.reshape(NC, EP // CH, CH)

    cnt = _hist(dst3)
    deg = jnp.concatenate([cnt[0, :NU], cnt[1, :NU]])
    dis = jnp.where(deg > 0, lax.rsqrt(deg), 0.0)
    dis2 = (dis * dis)[:, None]

    x0 = jnp.concatenate([user_emb, item_emb], axis=0)
    w = x0 * dis[:, None]
    ssum = jnp.zeros_like(x0)
    for l in range(3):
        o0, o1 = _layer(src3, dst3, w[:, :H], w[:, H:])
        sl = jnp.concatenate(
            [
                jnp.concatenate([o0[0, :NU], o0[1, :NU]], axis=0),
                jnp.concatenate([o1[0, :NU], o1[1, :NU]], axis=0),
            ],
            axis=1,
        )
        ssum = ssum + sl
        if l < 2:
            w = sl * dis2
    final = (x0 + ssum * dis[:, None]) * 0.25
    return final[:NU], final[NU:]


# speedup vs baseline: 15.5774x; 1.4328x over previous
import functools

import jax
import jax.numpy as jnp
from jax import lax
from jax.experimental import pallas as pl
from jax.experimental.pallas import tpu as pltpu
from jax.experimental.pallas import tpu_sc as plsc

NU = 50000
NT = 2 * NU
D = 64
H = 32
E = 800000
NC, NS, LN = 2, 16, 16
CH = 128
EP = 802816
G = EP // NS // CH
J = 8
NG = G // J
R = 51200
RT = R // NS
TRASH = R - 1
CL = 128
GL = EP // NS // CL
NBUF = 4

_mesh = plsc.VectorSubcoreMesh(
    core_axis_name="c", subcore_axis_name="s", num_cores=NC, num_subcores=NS
)
_params = pltpu.CompilerParams(use_tc_tiling_on_sc=False)


def _zero_f32(ref, nrows, ncols):
    z = jnp.zeros((LN,), jnp.float32)

    def body(r, _):
        for c0 in range(0, ncols, LN):
            ref[r, pl.ds(c0, LN)] = z
        return 0

    lax.fori_loop(0, nrows, body, 0)


def _zero_f32_1d(ref, n):
    z = jnp.zeros((LN,), jnp.float32)

    def body(r, _):
        ref[pl.ds(r * LN, LN)] = z
        return 0

    lax.fori_loop(0, n // LN, body, 0)


def _hist_body(dst3, cnt, didx, ones, zbuf, acc):
    c = lax.axis_index("c")
    s = lax.axis_index("s")
    one = jnp.ones((LN,), jnp.float32)
    for c0 in range(0, CH, LN):
        ones[pl.ds(c0, LN)] = one
    _zero_f32_1d(zbuf, RT)
    pltpu.sync_copy(zbuf, acc.at[pl.ds(s * RT, RT)])
    plsc.subcore_barrier()

    def group(g, _):
        base = s * G + g * J
        pltpu.sync_copy(dst3.at[c, pl.ds(base, J)], didx)
        for j in range(J):
            pltpu.sync_copy(ones, acc.at[didx.at[j]], add=True)
        return 0

    lax.fori_loop(0, NG, group, 0)
    plsc.subcore_barrier()
    pltpu.sync_copy(acc.at[pl.ds(s * RT, RT)], cnt.at[c, pl.ds(s * RT, RT)])


_hist = pl.kernel(
    _hist_body,
    out_type=jax.ShapeDtypeStruct((NC, R), jnp.float32),
    mesh=_mesh,
    scratch_types=[
        pltpu.VMEM((J, CH), jnp.int32),
        pltpu.VMEM((CH,), jnp.float32),
        pltpu.VMEM((RT,), jnp.float32),
        pltpu.VMEM_SHARED((R,), jnp.float32),
    ],
    compiler_params=_params,
)


def _layer_body(src3, dst3, y0, y1, o0, o1, sidx, didx, zbuf, acc,
                *rows_sems):
    rows = rows_sems[:NBUF]
    gsem = rows_sems[NBUF:]
    c = lax.axis_index("c")
    s = lax.axis_index("s")
    _zero_f32(zbuf, CL, H)
    for h, (y, o) in enumerate(((y0, o0), (y1, o1))):
        for k in range(RT // CL):
            pltpu.sync_copy(zbuf, acc.at[pl.ds(s * RT + k * CL, CL)])
        plsc.subcore_barrier()

        def group(g, _):
            base = s * G + g * J
            pltpu.sync_copy(src3.at[c, pl.ds(base, J)], sidx)
            pltpu.sync_copy(dst3.at[c, pl.ds(base, J)], didx)
            for b in range(NBUF):
                pltpu.async_copy(y.at[sidx.at[b]], rows[b], gsem[b])
            for j in range(J):
                b = j % NBUF
                pltpu.make_async_copy(y.at[sidx.at[j]], rows[b],
                                      gsem[b]).wait()
                pltpu.sync_copy(rows[b], acc.at[didx.at[j]], add=True)
                if j + NBUF < J:
                    pltpu.async_copy(y.at[sidx.at[j + NBUF]], rows[b],
                                     gsem[b])
            return 0

        lax.fori_loop(0, NG, group, 0)
        plsc.subcore_barrier()
        pltpu.sync_copy(
            acc.at[pl.ds(s * RT, RT)], o.at[c, pl.ds(s * RT, RT)]
        )
        if h == 0:
            plsc.subcore_barrier()


_layer = pl.kernel(
    _layer_body,
    out_type=[
        jax.ShapeDtypeStruct((NC, R, H), jnp.float32),
        jax.ShapeDtypeStruct((NC, R, H), jnp.float32),
    ],
    mesh=_mesh,
    scratch_types=[
        pltpu.VMEM((J, CL), jnp.int32),
        pltpu.VMEM((J, CL), jnp.int32),
        pltpu.VMEM((CL, H), jnp.float32),
        pltpu.VMEM_SHARED((R, H), jnp.float32),
    ]
    + [pltpu.VMEM((CL, H), jnp.float32)] * NBUF
    + [pltpu.SemaphoreType.DMA] * NBUF,
    compiler_params=_params,
)


@jax.jit
def kernel(edge_index, user_emb, item_emb):
    edge_index = edge_index.astype(jnp.int32)
    u, it = edge_index[0], edge_index[1]
    pad = EP - E
    dst = jnp.stack([u, it])
    src = jnp.stack([it + NU, u])
    dstp = jnp.pad(dst, ((0, 0), (0, pad)), constant_values=TRASH)
    srcp = jnp.pad(src, ((0, 0), (0, pad)), constant_values=0)
    dst3h = dstp.reshape(NC, EP // CH, CH)
    dst3 = dstp.reshape(NC, EP // CL, CL)
    src3 = srcp.reshape(NC, EP // CL, CL)

    cnt = _hist(dst3h)
    deg = jnp.concatenate([cnt[0, :NU], cnt[1, :NU]])
    dis = jnp.where(deg > 0, lax.rsqrt(deg), 0.0)
    dis2 = (dis * dis)[:, None]

    x0 = jnp.concatenate([user_emb, item_emb], axis=0)
    w = x0 * dis[:, None]
    ssum = jnp.zeros_like(x0)
    for l in range(3):
        o0, o1 = _layer(src3, dst3, w[:, :H], w[:, H:])
        sl = jnp.concatenate(
            [
                jnp.concatenate([o0[0, :NU], o0[1, :NU]], axis=0),
                jnp.concatenate([o1[0, :NU], o1[1, :NU]], axis=0),
            ],
            axis=1,
        )
        ssum = ssum + sl
        if l < 2:
            w = sl * dis2
    final = (x0 + ssum * dis[:, None]) * 0.25
    return final[:NU], final[NU:]


# speedup vs baseline: 17.3872x; 1.1162x over previous
import jax
import jax.numpy as jnp
from jax import lax
from jax.experimental import pallas as pl
from jax.experimental.pallas import tpu as pltpu
from jax.experimental.pallas import tpu_sc as plsc

NU = 50000
H = 32
E = 800000
NC, NS, LN = 2, 16, 16
CL = 128
EP = 802816
G = EP // NS // CL
J = 8
NG = G // J
R = 51200
RT = R // NS
TRASH = R - 1
NBUF = 4

_mesh = plsc.VectorSubcoreMesh(
    core_axis_name="c", subcore_axis_name="s", num_cores=NC, num_subcores=NS
)
_params = pltpu.CompilerParams(use_tc_tiling_on_sc=False)


def _zero_f32(ref, nrows, ncols):
    z = jnp.zeros((LN,), jnp.float32)

    def body(r, _):
        for c0 in range(0, ncols, LN):
            ref[r, pl.ds(c0, LN)] = z
        return 0

    lax.fori_loop(0, nrows, body, 0)


def _zero_f32_1d(ref, n):
    z = jnp.zeros((LN,), jnp.float32)

    def body(r, _):
        ref[pl.ds(r * LN, LN)] = z
        return 0

    lax.fori_loop(0, n // LN, body, 0)


def _hist_body(eidx, cnt, didx, ones, zbuf, acc):
    c = lax.axis_index("c")
    s = lax.axis_index("s")
    one = jnp.ones((LN,), jnp.float32)
    for c0 in range(0, CL, LN):
        ones[pl.ds(c0, LN)] = one
    _zero_f32_1d(zbuf, RT)
    pltpu.sync_copy(zbuf, acc.at[pl.ds(s * RT, RT)])
    plsc.subcore_barrier()

    def group(g, _):
        base = s * G + g * J
        pltpu.sync_copy(eidx.at[c, pl.ds(base, J)], didx)
        for j in range(J):
            pltpu.sync_copy(ones, acc.at[didx.at[j]], add=True)
        return 0

    lax.fori_loop(0, NG, group, 0)
    plsc.subcore_barrier()
    pltpu.sync_copy(acc.at[pl.ds(s * RT, RT)], cnt.at[c, pl.ds(s * RT, RT)])


_hist = pl.kernel(
    _hist_body,
    out_type=jax.ShapeDtypeStruct((NC, R), jnp.float32),
    mesh=_mesh,
    scratch_types=[
        pltpu.VMEM((J, CL), jnp.int32),
        pltpu.VMEM((CL,), jnp.float32),
        pltpu.VMEM((RT,), jnp.float32),
        pltpu.VMEM_SHARED((R,), jnp.float32),
    ],
    compiler_params=_params,
)


def _layer_body(eidx, w0, w1, o0, o1, sidx, didx, zbuf, acc, *rows_sems):
    rows = rows_sems[:NBUF]
    gsem = rows_sems[NBUF:]
    c = lax.axis_index("c")
    s = lax.axis_index("s")
    notc = 1 - c
    _zero_f32(zbuf, CL, H)
    for h, (w, o) in enumerate(((w0, o0), (w1, o1))):
        ytab = w.at[notc]
        for k in range(RT // CL):
            pltpu.sync_copy(zbuf, acc.at[pl.ds(s * RT + k * CL, CL)])
        plsc.subcore_barrier()

        def group(g, _):
            base = s * G + g * J
            pltpu.sync_copy(eidx.at[notc, pl.ds(base, J)], sidx)
            pltpu.sync_copy(eidx.at[c, pl.ds(base, J)], didx)
            for b in range(NBUF):
                pltpu.async_copy(ytab.at[sidx.at[b]], rows[b], gsem[b])
            for j in range(J):
                b = j % NBUF
                pltpu.make_async_copy(ytab.at[sidx.at[j]], rows[b],
                                      gsem[b]).wait()
                pltpu.sync_copy(rows[b], acc.at[didx.at[j]], add=True)
                if j + NBUF < J:
                    pltpu.async_copy(ytab.at[sidx.at[j + NBUF]], rows[b],
                                     gsem[b])
            return 0

        lax.fori_loop(0, NG, group, 0)
        plsc.subcore_barrier()
        pltpu.sync_copy(
            acc.at[pl.ds(s * RT, RT)], o.at[c, pl.ds(s * RT, RT)]
        )
        if h == 0:
            plsc.subcore_barrier()


_layer = pl.kernel(
    _layer_body,
    out_type=[
        jax.ShapeDtypeStruct((NC, R, H), jnp.float32),
        jax.ShapeDtypeStruct((NC, R, H), jnp.float32),
    ],
    mesh=_mesh,
    scratch_types=[
        pltpu.VMEM((J, CL), jnp.int32),
        pltpu.VMEM((J, CL), jnp.int32),
        pltpu.VMEM((CL, H), jnp.float32),
        pltpu.VMEM_SHARED((R, H), jnp.float32),
    ]
    + [pltpu.VMEM((CL, H), jnp.float32)] * NBUF
    + [pltpu.SemaphoreType.DMA] * NBUF,
    compiler_params=_params,
)


@jax.jit
def kernel(edge_index, user_emb, item_emb):
    ei = edge_index.astype(jnp.int32)
    eidx = jnp.pad(ei, ((0, 0), (0, EP - E)), constant_values=TRASH)
    eidx = eidx.reshape(NC, EP // CL, CL)

    cnt = _hist(eidx)
    degu, degi = cnt[0, :NU], cnt[1, :NU]
    disu = jnp.where(degu > 0, lax.rsqrt(degu), 0.0)[:, None]
    disi = jnp.where(degi > 0, lax.rsqrt(degi), 0.0)[:, None]
    d2pad = (
        jnp.zeros((NC, R, 1), jnp.float32)
        .at[0, :NU].set(disu * disu)
        .at[1, :NU].set(disi * disi)
    )

    xu = user_emb * disu
    xi = item_emb * disi
    ztab = jnp.zeros((NC, R, H), jnp.float32)
    w0 = ztab.at[0, :NU].set(xu[:, :H]).at[1, :NU].set(xi[:, :H])
    w1 = ztab.at[0, :NU].set(xu[:, H:]).at[1, :NU].set(xi[:, H:])

    s0 = jnp.zeros((NC, R, H), jnp.float32)
    s1 = jnp.zeros((NC, R, H), jnp.float32)
    for l in range(3):
        o0, o1 = _layer(eidx, w0, w1)
        s0 = s0 + o0
        s1 = s1 + o1
        if l < 2:
            w0 = o0 * d2pad
            w1 = o1 * d2pad

    user_final = (
        user_emb + disu * jnp.concatenate([s0[0, :NU], s1[0, :NU]], axis=1)
    ) * 0.25
    item_final = (
        item_emb + disi * jnp.concatenate([s0[1, :NU], s1[1, :NU]], axis=1)
    ) * 0.25
    return user_final, item_final
